# baseline (speedup 1.0000x reference)
import functools

import jax
import jax.numpy as jnp
from jax import lax
from jax.experimental import pallas as pl
from jax.experimental.pallas import tpu as pltpu
from jax.experimental.pallas import tpu_sc as plsc

_NC = 2
_NS = 16
_NW = _NC * _NS


def _sc_gather(table, idx_list):
    (n_rows, width) = table.shape
    e_total = idx_list[0].shape[0]
    epw = e_total // _NW
    n_idx = len(idx_list)
    mesh = plsc.VectorSubcoreMesh(core_axis_name="c", subcore_axis_name="s")

    @functools.partial(
        pl.kernel,
        out_type=[jax.ShapeDtypeStruct((e_total, width), jnp.float32)] * n_idx,
        mesh=mesh,
        scratch_types=[
            pltpu.VMEM((epw,), jnp.int32),
            pltpu.VMEM((epw, width), jnp.float32),
            pltpu.SemaphoreType.DMA,
        ],
        compiler_params=pltpu.CompilerParams(use_tc_tiling_on_sc=False),
    )
    def k(*refs):
        table_hbm = refs[0]
        idx_hbms = refs[1:1 + n_idx]
        out_hbms = refs[1 + n_idx:1 + 2 * n_idx]
        idx_v, rows_v, sem = refs[1 + 2 * n_idx:]
        c = lax.axis_index("c")
        s = lax.axis_index("s")
        base = (s * _NC + c) * epw
        for j in range(n_idx):
            pltpu.sync_copy(idx_hbms[j].at[pl.ds(base, epw)], idx_v)
            pltpu.async_copy(table_hbm.at[idx_v], rows_v, sem).wait()
            pltpu.sync_copy(rows_v, out_hbms[j].at[pl.ds(base, epw)])

    return list(k(table, *idx_list))


def _sc_scatter_add(msg, col, zeros_init):
    e_total = msg.shape[0]
    n_rows, width = zeros_init.shape
    epw = e_total // _NW
    rows_per_tile = n_rows // _NS
    mesh = plsc.VectorSubcoreMesh(core_axis_name="c", subcore_axis_name="s")

    @functools.partial(
        pl.kernel,
        out_type=jax.ShapeDtypeStruct((_NC * n_rows, width), jnp.float32),
        mesh=mesh,
        scratch_types=[
            pltpu.VMEM((epw,), jnp.int32),
            pltpu.VMEM((epw, width), jnp.float32),
            pltpu.VMEM_SHARED((n_rows, width), jnp.float32),
            pltpu.SemaphoreType.DMA,
        ],
        compiler_params=pltpu.CompilerParams(use_tc_tiling_on_sc=False),
    )
    def k(msg_hbm, col_hbm, zero_hbm, out_hbm, idx_v, msg_v, shared, sem):
        c = lax.axis_index("c")
        s = lax.axis_index("s")
        base = (c * _NS + s) * epw

        @pl.when(s == 0)
        def _():
            pltpu.sync_copy(zero_hbm, shared)

        plsc.subcore_barrier()
        pltpu.sync_copy(col_hbm.at[pl.ds(base, epw)], idx_v)
        pltpu.sync_copy(msg_hbm.at[pl.ds(base, epw)], msg_v)
        pltpu.sync_copy(msg_v, shared.at[idx_v], add=True)
        plsc.subcore_barrier()
        pltpu.sync_copy(
            shared.at[pl.ds(s * rows_per_tile, rows_per_tile)],
            out_hbm.at[pl.ds(c * n_rows + s * rows_per_tile, rows_per_tile)],
        )

    return k(msg, col, zeros_init)


def _tc_h0(x_p, w1big, b1rep):
    gn = x_p.shape[0]

    def body(x_ref, w_ref, b_ref, o_ref):
        o_ref[...] = jax.nn.silu(
            jnp.dot(x_ref[...], w_ref[...], preferred_element_type=jnp.float32)
            + b_ref[...])

    return pl.pallas_call(
        body,
        out_shape=jax.ShapeDtypeStruct((gn, 128), jnp.float32),
    )(x_p, w1big, b1rep)


def _tc_z(pr_p, pc_p, ea_p, sg, p8, a1bd, a1rep):
    g_total = pr_p.shape[0]
    blk = 2000

    def body(pr_ref, pc_ref, ea_ref, sg_ref, p8_ref, a1_ref, a1b_ref, o_ref):
        d = pr_ref[...] - pc_ref[...]
        t8 = jnp.dot(d * d, sg_ref[...], preferred_element_type=jnp.float32)
        dist8 = jnp.sqrt(t8 + 1e-12)
        e_in = ea_ref[...] + jnp.dot(dist8, p8_ref[...],
                                     preferred_element_type=jnp.float32)
        o_ref[...] = jax.nn.silu(
            jnp.dot(e_in, a1_ref[...], preferred_element_type=jnp.float32)
            + a1b_ref[...])

    return pl.pallas_call(
        body,
        grid=(g_total // blk,),
        in_specs=[
            pl.BlockSpec((blk, 128), lambda i: (i, 0)),
            pl.BlockSpec((blk, 128), lambda i: (i, 0)),
            pl.BlockSpec((blk, 128), lambda i: (i, 0)),
            pl.BlockSpec((128, 8), lambda i: (0, 0)),
            pl.BlockSpec((8, 128), lambda i: (0, 0)),
            pl.BlockSpec((128, 256), lambda i: (0, 0)),
            pl.BlockSpec((1, 256), lambda i: (0, 0)),
        ],
        out_specs=pl.BlockSpec((blk, 256), lambda i: (i, 0)),
        out_shape=jax.ShapeDtypeStruct((g_total, 256), jnp.float32),
    )(pr_p, pc_p, ea_p, sg, p8, a1bd, a1rep)


def _tc_msg(z_p, u_p, a2bd4, a2rep, tbd, sbd):
    g_total = u_p.shape[0]
    blk = 400

    def body(z_ref, u_ref, a2_ref, a2b_ref, t_ref, s_ref, o_ref):
        zv = z_ref[...].astype(jnp.bfloat16)
        a2v = a2_ref[...].astype(jnp.bfloat16)
        wep_a = jnp.dot(zv[:, :128], a2v, preferred_element_type=jnp.float32)
        wep_b = jnp.dot(zv[:, 128:], a2v, preferred_element_type=jnp.float32)
        wep = jnp.concatenate([wep_a, wep_b], axis=1) + a2b_ref[...]
        uexp = jnp.dot(u_ref[...].astype(jnp.bfloat16),
                       t_ref[...].astype(jnp.bfloat16),
                       preferred_element_type=jnp.float32)
        prod = (wep * uexp).astype(jnp.bfloat16)
        o_ref[...] = jnp.dot(prod, s_ref[...].astype(jnp.bfloat16),
                             preferred_element_type=jnp.float32)

    return pl.pallas_call(
        body,
        grid=(g_total // blk,),
        in_specs=[
            pl.BlockSpec((blk, 256), lambda i: (i, 0)),
            pl.BlockSpec((blk, 128), lambda i: (i, 0)),
            pl.BlockSpec((128, 1024), lambda i: (0, 0)),
            pl.BlockSpec((1, 2048), lambda i: (0, 0)),
            pl.BlockSpec((128, 2048), lambda i: (0, 0)),
            pl.BlockSpec((2048, 128), lambda i: (0, 0)),
        ],
        out_specs=pl.BlockSpec((blk, 128), lambda i: (i, 0)),
        out_shape=jax.ShapeDtypeStruct((g_total, 128), jnp.float32),
    )(z_p, u_p, a2bd4, a2rep, tbd, sbd)


def _tc_gru(h_p, parts_p, wr_bd, br_rep, wih_big, whh_big, bih_big, bhh_big):
    gn = h_p.shape[0]

    def body(h_ref, p_ref, wr_ref, br_ref, wih_ref, whh_ref, bih_ref,
             bhh_ref, o_ref):
        hv = h_ref[...]
        aggr = p_ref[:gn, :] + p_ref[gn:, :]
        hc = jax.nn.silu(
            jnp.dot(hv, wr_ref[...], preferred_element_type=jnp.float32)
            + br_ref[...] + aggr)
        gi = jnp.dot(hc, wih_ref[...], preferred_element_type=jnp.float32) + bih_ref[...]
        gh = jnp.dot(hv, whh_ref[...], preferred_element_type=jnp.float32) + bhh_ref[...]
        r = jax.nn.sigmoid(gi[:, 0:128] + gh[:, 0:128])
        z = jax.nn.sigmoid(gi[:, 128:256] + gh[:, 128:256])
        nn = jnp.tanh(gi[:, 256:384] + r * gh[:, 256:384])
        o_ref[...] = (1.0 - z) * nn + z * hv

    return pl.pallas_call(
        body,
        out_shape=jax.ShapeDtypeStruct((gn, 128), jnp.float32),
    )(h_p, parts_p, wr_bd, br_rep, wih_big, whh_big, bih_big, bhh_big)


def _tc_set2set(h_state, batch_col, wi_t, wh_t, bi_row, bh_row,
                o1m, o1b, o2m, o2b, m_steps, n_graphs):
    n, h = h_state.shape

    def body(h_ref, b_ref, wi_ref, wh_ref, bi_ref, bh_ref,
             o1_ref, o1b_ref, o2_ref, o2b_ref, out_ref):
        hv = h_ref[...]
        seg = b_ref[...]
        gid = lax.broadcasted_iota(jnp.int32, (n, n_graphs), 1)
        mask = seg == gid
        onehot = mask.astype(jnp.float32)

        q_star = jnp.zeros((n_graphs, 2 * h), jnp.float32)
        lh = jnp.zeros((n_graphs, h), jnp.float32)
        lc = jnp.zeros((n_graphs, h), jnp.float32)
        for _ in range(m_steps):
            g = (jnp.dot(q_star, wi_ref[...], preferred_element_type=jnp.float32)
                 + bi_ref[...]
                 + jnp.dot(lh, wh_ref[...], preferred_element_type=jnp.float32)
                 + bh_ref[...])
            i_ = jax.nn.sigmoid(g[:, 0:h])
            f_ = jax.nn.sigmoid(g[:, h:2 * h])
            c_ = jnp.tanh(g[:, 2 * h:3 * h])
            o_ = jax.nn.sigmoid(g[:, 3 * h:4 * h])
            lc = f_ * lc + i_ * c_
            lh = o_ * jnp.tanh(lc)
            q = lh
            qpn = jnp.dot(onehot, q, preferred_element_type=jnp.float32)
            e = jnp.sum(hv * qpn, axis=1, keepdims=True)
            me = jnp.where(mask, e, -3e38)
            emax = jnp.max(me, axis=0, keepdims=True)
            emax = jnp.where(emax > -1e38, emax, 0.0)
            eg = jnp.sum(onehot * emax, axis=1, keepdims=True)
            ee = jnp.exp(e - eg)
            den = jnp.sum(onehot * ee, axis=0, keepdims=True)
            dg = jnp.sum(onehot * den, axis=1, keepdims=True)
            a = ee / (dg + 1e-16)
            wgt = onehot * a
            rvec = lax.dot_general(wgt, hv, (((0,), (0,)), ((), ())),
                                   preferred_element_type=jnp.float32)
            q_star = jnp.concatenate([q, rvec], axis=1)

        s = jax.nn.silu(
            jnp.dot(q_star, o1_ref[...], preferred_element_type=jnp.float32)
            + o1b_ref[...])
        out_ref[...] = (jnp.dot(s, o2_ref[...], preferred_element_type=jnp.float32)
                        + o2b_ref[...])

    return pl.pallas_call(
        body,
        out_shape=jax.ShapeDtypeStruct((n_graphs, 1), jnp.float32),
    )(h_state, batch_col, wi_t, wh_t, bi_row, bh_row, o1m, o1b, o2m, o2b)


def _kron8(m):
    return jnp.kron(jnp.eye(8, dtype=jnp.float32), m)


def kernel(x, edge_index, edge_attr, pos, batch, W1, b1, A1, a1, A2, a2,
           Wr, br, Wih, Whh, bih, bhh, Wi, Wh, bi, bh, O1, o1, O2, o2):
    n, d_in = x.shape
    e_total = edge_index.shape[1]
    h = W1.shape[1]
    m_steps = 3
    num_layers = 4
    b_graphs = 16
    ge = e_total // 8
    gn = n // 8

    row = edge_index[0]
    col = edge_index[1]

    eye16 = jnp.eye(16, dtype=jnp.float32)
    sg = _kron8(jnp.ones((16, 1), jnp.float32))
    p8 = _kron8(jnp.zeros((1, 16), jnp.float32).at[0, 4].set(1.0))
    a1p16 = jnp.pad(A1, ((0, 11), (0, 0)))
    a1bd = _kron8(a1p16)
    a1rep = jnp.tile(a1.reshape(1, 2 * h), (1, 8))
    a2perm = A2.reshape(2 * h, h, h).transpose(0, 2, 1).reshape(2 * h, h * h)
    a2bd4 = jnp.kron(jnp.eye(4, dtype=jnp.float32), a2perm)
    a2rep = jnp.tile(a2.reshape(h, h).T.reshape(1, h * h), (1, 8))
    t16 = jnp.tile(eye16, (1, 16))
    tbd = jnp.zeros((128, 2048), jnp.float32)
    sbd = jnp.zeros((2048, 128), jnp.float32)
    s16 = jnp.repeat(eye16, 16, axis=0)
    for j in range(8):
        tbd = tbd.at[16 * j:16 * (j + 1), 256 * j:256 * (j + 1)].set(t16)
        sbd = sbd.at[256 * j:256 * (j + 1), 16 * j:16 * (j + 1)].set(s16)
    w1big = _kron8(W1)
    b1rep = jnp.tile(b1.reshape(1, h), (1, 8))
    wr_bd = _kron8(Wr)
    br_rep = jnp.tile(br.reshape(1, h), (1, 8))
    wih_t = Wih.T
    whh_t = Whh.T
    wih_big = jnp.zeros((128, 384), jnp.float32)
    whh_big = jnp.zeros((128, 384), jnp.float32)
    for j in range(8):
        for g in range(3):
            wih_big = wih_big.at[16 * j:16 * (j + 1),
                                 128 * g + 16 * j:128 * g + 16 * (j + 1)].set(
                wih_t[:, 16 * g:16 * (g + 1)])
            whh_big = whh_big.at[16 * j:16 * (j + 1),
                                 128 * g + 16 * j:128 * g + 16 * (j + 1)].set(
                whh_t[:, 16 * g:16 * (g + 1)])
    bih_big = jnp.concatenate(
        [jnp.tile(bih[16 * g:16 * (g + 1)].reshape(1, 16), (1, 8))
         for g in range(3)], axis=1)
    bhh_big = jnp.concatenate(
        [jnp.tile(bhh[16 * g:16 * (g + 1)].reshape(1, 16), (1, 8))
         for g in range(3)], axis=1)

    pos16 = jnp.zeros((n, 16), jnp.float32).at[:, :3].set(pos)
    ea_p = jnp.pad(edge_attr, ((0, 0), (0, 12))).reshape(ge, 128)
    zeros_init = jnp.zeros((n, h), jnp.float32)
    x_p = x.reshape(gn, 8 * d_in)

    pr, pc = _sc_gather(pos16, [row, col])
    z_p = _tc_z(pr.reshape(ge, 128), pc.reshape(ge, 128), ea_p,
                sg, p8, a1bd, a1rep)
    h_p = _tc_h0(x_p, w1big, b1rep)

    for _ in range(num_layers):
        (u,) = _sc_gather(h_p.reshape(n, h), [row])
        msg_p = _tc_msg(z_p, u.reshape(ge, 128), a2bd4, a2rep, tbd, sbd)
        parts = _sc_scatter_add(msg_p.reshape(e_total, h), col, zeros_init)
        h_p = _tc_gru(h_p, parts.reshape(2 * gn, 128), wr_bd, br_rep,
                      wih_big, whh_big, bih_big, bhh_big)

    out = _tc_set2set(h_p.reshape(n, h), batch.reshape(n, 1), Wi.T, Wh.T,
                      bi.reshape(1, 4 * h), bh.reshape(1, 4 * h),
                      O1, o1.reshape(1, h), O2, o2.reshape(1, 1),
                      m_steps, b_graphs)
    return jnp.squeeze(out)


# speedup vs baseline: 1.1605x; 1.1605x over previous
import functools

import jax
import jax.numpy as jnp
import numpy as np
from jax import lax
from jax.experimental import pallas as pl
from jax.experimental.pallas import tpu as pltpu
from jax.experimental.pallas import tpu_sc as plsc

_NC = 2
_NS = 16
_NW = _NC * _NS


def _sc_gather(table, idx_list):
    (n_rows, width) = table.shape
    e_total = idx_list[0].shape[0]
    epw = e_total // _NW
    n_idx = len(idx_list)
    mesh = plsc.VectorSubcoreMesh(core_axis_name="c", subcore_axis_name="s")

    @functools.partial(
        pl.kernel,
        out_type=[jax.ShapeDtypeStruct((e_total, width), jnp.float32)] * n_idx,
        mesh=mesh,
        scratch_types=[
            pltpu.VMEM((epw,), jnp.int32),
            pltpu.VMEM((epw, width), jnp.float32),
            pltpu.SemaphoreType.DMA,
        ],
        compiler_params=pltpu.CompilerParams(use_tc_tiling_on_sc=False),
    )
    def k(*refs):
        table_hbm = refs[0]
        idx_hbms = refs[1:1 + n_idx]
        out_hbms = refs[1 + n_idx:1 + 2 * n_idx]
        idx_v, rows_v, sem = refs[1 + 2 * n_idx:]
        c = lax.axis_index("c")
        s = lax.axis_index("s")
        base = (s * _NC + c) * epw
        for j in range(n_idx):
            pltpu.sync_copy(idx_hbms[j].at[pl.ds(base, epw)], idx_v)
            pltpu.async_copy(table_hbm.at[idx_v], rows_v, sem).wait()
            pltpu.sync_copy(rows_v, out_hbms[j].at[pl.ds(base, epw)])

    return list(k(table, *idx_list))


def _sc_scatter_add(msg, col, zeros_init):
    e_total = msg.shape[0]
    n_rows, width = zeros_init.shape
    epw = e_total // _NW
    rows_per_tile = n_rows // _NS
    mesh = plsc.VectorSubcoreMesh(core_axis_name="c", subcore_axis_name="s")

    @functools.partial(
        pl.kernel,
        out_type=jax.ShapeDtypeStruct((_NC * n_rows, width), jnp.float32),
        mesh=mesh,
        scratch_types=[
            pltpu.VMEM((epw,), jnp.int32),
            pltpu.VMEM((epw, width), jnp.float32),
            pltpu.VMEM_SHARED((n_rows, width), jnp.float32),
            pltpu.SemaphoreType.DMA,
        ],
        compiler_params=pltpu.CompilerParams(use_tc_tiling_on_sc=False),
    )
    def k(msg_hbm, col_hbm, zero_hbm, out_hbm, idx_v, msg_v, shared, sem):
        c = lax.axis_index("c")
        s = lax.axis_index("s")
        base = (c * _NS + s) * epw

        @pl.when(s == 0)
        def _():
            pltpu.sync_copy(zero_hbm, shared)

        plsc.subcore_barrier()
        pltpu.sync_copy(col_hbm.at[pl.ds(base, epw)], idx_v)
        pltpu.sync_copy(msg_hbm.at[pl.ds(base, epw)], msg_v)
        pltpu.sync_copy(msg_v, shared.at[idx_v], add=True)
        plsc.subcore_barrier()
        pltpu.sync_copy(
            shared.at[pl.ds(s * rows_per_tile, rows_per_tile)],
            out_hbm.at[pl.ds(c * n_rows + s * rows_per_tile, rows_per_tile)],
        )

    return k(msg, col, zeros_init)


def _tc_h0(x_p, w1big, b1rep):
    gn = x_p.shape[0]

    def body(x_ref, w_ref, b_ref, o_ref):
        o_ref[...] = jax.nn.silu(
            jnp.dot(x_ref[...], w_ref[...], preferred_element_type=jnp.float32)
            + b_ref[...])

    return pl.pallas_call(
        body,
        out_shape=jax.ShapeDtypeStruct((gn, 128), jnp.float32),
    )(x_p, w1big, b1rep)


def _tc_z(pr_p, pc_p, ea32, sg, c_ea, c_dist, a1rep):
    g_total = pr_p.shape[0]
    blk = 2000

    def body(pr_ref, pc_ref, ea_ref, sg_ref, cea_ref, cd_ref, a1b_ref, o_ref):
        d = pr_ref[...] - pc_ref[...]
        t8 = jnp.dot(d * d, sg_ref[...], preferred_element_type=jnp.float32)
        dist8 = jnp.sqrt(t8 + 1e-12)
        pre = (jnp.dot(ea_ref[...], cea_ref[...],
                       preferred_element_type=jnp.float32)
               + jnp.dot(dist8, cd_ref[...],
                         preferred_element_type=jnp.float32)
               + a1b_ref[...])
        o_ref[...] = jax.nn.silu(pre)

    return pl.pallas_call(
        body,
        grid=(g_total // blk,),
        in_specs=[
            pl.BlockSpec((blk, 128), lambda i: (i, 0)),
            pl.BlockSpec((blk, 128), lambda i: (i, 0)),
            pl.BlockSpec((blk, 32), lambda i: (i, 0)),
            pl.BlockSpec((128, 8), lambda i: (0, 0)),
            pl.BlockSpec((32, 256), lambda i: (0, 0)),
            pl.BlockSpec((8, 256), lambda i: (0, 0)),
            pl.BlockSpec((1, 256), lambda i: (0, 0)),
        ],
        out_specs=pl.BlockSpec((blk, 256), lambda i: (i, 0)),
        out_shape=jax.ShapeDtypeStruct((g_total, 256), jnp.float32),
    )(pr_p, pc_p, ea32, sg, c_ea, c_dist, a1rep)


def _tc_msg(z_p, u_p, a2bd4, a2rep, tbd, sbd):
    g_total = u_p.shape[0]
    blk = 400

    def body(z_ref, u_ref, a2_ref, a2b_ref, t_ref, s_ref, o_ref):
        zv = z_ref[...]
        wep_a = jnp.dot(zv[:, :128], a2_ref[...],
                        preferred_element_type=jnp.float32)
        wep_b = jnp.dot(zv[:, 128:], a2_ref[...],
                        preferred_element_type=jnp.float32)
        wep = jnp.concatenate([wep_a, wep_b], axis=1) + a2b_ref[...]
        uexp = jnp.dot(u_ref[...], t_ref[...],
                       preferred_element_type=jnp.float32)
        o_ref[...] = jnp.dot(wep * uexp, s_ref[...],
                             preferred_element_type=jnp.float32)

    return pl.pallas_call(
        body,
        grid=(g_total // blk,),
        in_specs=[
            pl.BlockSpec((blk, 256), lambda i: (i, 0)),
            pl.BlockSpec((blk, 128), lambda i: (i, 0)),
            pl.BlockSpec((128, 1024), lambda i: (0, 0)),
            pl.BlockSpec((1, 2048), lambda i: (0, 0)),
            pl.BlockSpec((128, 2048), lambda i: (0, 0)),
            pl.BlockSpec((2048, 128), lambda i: (0, 0)),
        ],
        out_specs=pl.BlockSpec((blk, 128), lambda i: (i, 0)),
        out_shape=jax.ShapeDtypeStruct((g_total, 128), jnp.float32),
    )(z_p, u_p, a2bd4, a2rep, tbd, sbd)


def _tc_gru(h_p, parts_p, wr_bd, br_rep, wih_big, whh_big, bih_big, bhh_big):
    gn = h_p.shape[0]

    def body(h_ref, p_ref, wr_ref, br_ref, wih_ref, whh_ref, bih_ref,
             bhh_ref, o_ref):
        hv = h_ref[...]
        aggr = p_ref[:gn, :] + p_ref[gn:, :]
        hc = jax.nn.silu(
            jnp.dot(hv, wr_ref[...], preferred_element_type=jnp.float32)
            + br_ref[...] + aggr)
        gi = jnp.dot(hc, wih_ref[...], preferred_element_type=jnp.float32) + bih_ref[...]
        gh = jnp.dot(hv, whh_ref[...], preferred_element_type=jnp.float32) + bhh_ref[...]
        r = jax.nn.sigmoid(gi[:, 0:128] + gh[:, 0:128])
        z = jax.nn.sigmoid(gi[:, 128:256] + gh[:, 128:256])
        nn = jnp.tanh(gi[:, 256:384] + r * gh[:, 256:384])
        o_ref[...] = (1.0 - z) * nn + z * hv

    return pl.pallas_call(
        body,
        out_shape=jax.ShapeDtypeStruct((gn, 128), jnp.float32),
    )(h_p, parts_p, wr_bd, br_rep, wih_big, whh_big, bih_big, bhh_big)


def _tc_set2set(h_state, batch_col, wi_t, wh_t, bi_row, bh_row,
                o1m, o1b, o2m, o2b, m_steps, n_graphs):
    n, h = h_state.shape

    def body(h_ref, b_ref, wi_ref, wh_ref, bi_ref, bh_ref,
             o1_ref, o1b_ref, o2_ref, o2b_ref, out_ref):
        hv = h_ref[...]
        seg = b_ref[...]
        gid = lax.broadcasted_iota(jnp.int32, (n, n_graphs), 1)
        mask = seg == gid
        onehot = mask.astype(jnp.float32)

        q_star = jnp.zeros((n_graphs, 2 * h), jnp.float32)
        lh = jnp.zeros((n_graphs, h), jnp.float32)
        lc = jnp.zeros((n_graphs, h), jnp.float32)
        for _ in range(m_steps):
            g = (jnp.dot(q_star, wi_ref[...], preferred_element_type=jnp.float32)
                 + bi_ref[...]
                 + jnp.dot(lh, wh_ref[...], preferred_element_type=jnp.float32)
                 + bh_ref[...])
            i_ = jax.nn.sigmoid(g[:, 0:h])
            f_ = jax.nn.sigmoid(g[:, h:2 * h])
            c_ = jnp.tanh(g[:, 2 * h:3 * h])
            o_ = jax.nn.sigmoid(g[:, 3 * h:4 * h])
            lc = f_ * lc + i_ * c_
            lh = o_ * jnp.tanh(lc)
            q = lh
            qpn = jnp.dot(onehot, q, preferred_element_type=jnp.float32)
            e = jnp.sum(hv * qpn, axis=1, keepdims=True)
            me = jnp.where(mask, e, -3e38)
            emax = jnp.max(me, axis=0, keepdims=True)
            emax = jnp.where(emax > -1e38, emax, 0.0)
            eg = jnp.sum(onehot * emax, axis=1, keepdims=True)
            ee = jnp.exp(e - eg)
            den = jnp.sum(onehot * ee, axis=0, keepdims=True)
            dg = jnp.sum(onehot * den, axis=1, keepdims=True)
            a = ee / (dg + 1e-16)
            wgt = onehot * a
            rvec = lax.dot_general(wgt, hv, (((0,), (0,)), ((), ())),
                                   preferred_element_type=jnp.float32)
            q_star = jnp.concatenate([q, rvec], axis=1)

        s = jax.nn.silu(
            jnp.dot(q_star, o1_ref[...], preferred_element_type=jnp.float32)
            + o1b_ref[...])
        out_ref[...] = (jnp.dot(s, o2_ref[...], preferred_element_type=jnp.float32)
                        + o2b_ref[...])

    return pl.pallas_call(
        body,
        out_shape=jax.ShapeDtypeStruct((n_graphs, 1), jnp.float32),
    )(h_state, batch_col, wi_t, wh_t, bi_row, bh_row, o1m, o1b, o2m, o2b)


def kernel(x, edge_index, edge_attr, pos, batch, W1, b1, A1, a1, A2, a2,
           Wr, br, Wih, Whh, bih, bhh, Wi, Wh, bi, bh, O1, o1, O2, o2):
    n, d_in = x.shape
    e_total = edge_index.shape[1]
    h = W1.shape[1]
    m_steps = 3
    num_layers = 4
    b_graphs = 16
    ge = e_total // 8
    gn = n // 8

    row = edge_index[0]
    col = edge_index[1]

    eye8 = np.eye(8, dtype=np.float32)
    eye16 = np.eye(16, dtype=np.float32)
    sg = jnp.asarray(np.kron(eye8, np.ones((16, 1), np.float32)))
    c_ea = jnp.kron(eye8, A1[:4])
    c_dist = jnp.kron(eye8, A1[4:5])
    a1rep = jnp.tile(a1.reshape(1, 2 * h), (1, 8))
    a2perm = A2.reshape(2 * h, h, h).transpose(0, 2, 1).reshape(2 * h, h * h)
    a2bd4 = jnp.kron(jnp.eye(4, dtype=jnp.float32), a2perm)
    a2rep = jnp.tile(a2.reshape(h, h).T.reshape(1, h * h), (1, 8))
    t16 = np.tile(eye16, (1, 16))
    s16 = np.repeat(eye16, 16, axis=0)
    tbd = jnp.asarray(np.kron(eye8, t16))
    sbd = jnp.asarray(np.kron(eye8, s16))
    w1big = jnp.kron(jnp.eye(8, dtype=jnp.float32), W1)
    b1rep = jnp.tile(b1.reshape(1, h), (1, 8))
    wr_bd = jnp.kron(jnp.eye(8, dtype=jnp.float32), Wr)
    br_rep = jnp.tile(br.reshape(1, h), (1, 8))
    wih_t = Wih.T
    whh_t = Whh.T
    wih_big = jnp.concatenate(
        [jnp.kron(jnp.eye(8, dtype=jnp.float32),
                  wih_t[:, 16 * g:16 * (g + 1)]) for g in range(3)], axis=1)
    whh_big = jnp.concatenate(
        [jnp.kron(jnp.eye(8, dtype=jnp.float32),
                  whh_t[:, 16 * g:16 * (g + 1)]) for g in range(3)], axis=1)
    bih_big = jnp.concatenate(
        [jnp.tile(bih[16 * g:16 * (g + 1)].reshape(1, 16), (1, 8))
         for g in range(3)], axis=1)
    bhh_big = jnp.concatenate(
        [jnp.tile(bhh[16 * g:16 * (g + 1)].reshape(1, 16), (1, 8))
         for g in range(3)], axis=1)

    pos16 = jnp.zeros((n, 16), jnp.float32).at[:, :3].set(pos)
    ea32 = edge_attr.reshape(ge, 32)
    zeros_init = jnp.zeros((n, h), jnp.float32)
    x_p = x.reshape(gn, 8 * d_in)

    pr, pc = _sc_gather(pos16, [row, col])
    z_p = _tc_z(pr.reshape(ge, 128), pc.reshape(ge, 128), ea32,
                sg, c_ea, c_dist, a1rep)
    h_p = _tc_h0(x_p, w1big, b1rep)

    for _ in range(num_layers):
        (u,) = _sc_gather(h_p.reshape(n, h), [row])
        msg_p = _tc_msg(z_p, u.reshape(ge, 128), a2bd4, a2rep, tbd, sbd)
        parts = _sc_scatter_add(msg_p.reshape(e_total, h), col, zeros_init)
        h_p = _tc_gru(h_p, parts.reshape(2 * gn, 128), wr_bd, br_rep,
                      wih_big, whh_big, bih_big, bhh_big)

    out = _tc_set2set(h_p.reshape(n, h), batch.reshape(n, 1), Wi.T, Wh.T,
                      bi.reshape(1, 4 * h), bh.reshape(1, 4 * h),
                      O1, o1.reshape(1, h), O2, o2.reshape(1, 1),
                      m_steps, b_graphs)
    return jnp.squeeze(out)


# speedup vs baseline: 1.3752x; 1.1850x over previous
import functools

import jax
import jax.numpy as jnp
import numpy as np
from jax import lax
from jax.experimental import pallas as pl
from jax.experimental.pallas import tpu as pltpu
from jax.experimental.pallas import tpu_sc as plsc

_NC = 2
_NS = 16
_NW = _NC * _NS


def _sc_gather(table, idx_list):
    (n_rows, width) = table.shape
    e_total = idx_list[0].shape[0]
    epw = e_total // _NW
    n_idx = len(idx_list)
    mesh = plsc.VectorSubcoreMesh(core_axis_name="c", subcore_axis_name="s")

    @functools.partial(
        pl.kernel,
        out_type=[jax.ShapeDtypeStruct((e_total, width), jnp.float32)] * n_idx,
        mesh=mesh,
        scratch_types=[
            pltpu.VMEM((epw,), jnp.int32),
            pltpu.VMEM((epw, width), jnp.float32),
            pltpu.SemaphoreType.DMA,
        ],
        compiler_params=pltpu.CompilerParams(use_tc_tiling_on_sc=False),
    )
    def k(*refs):
        table_hbm = refs[0]
        idx_hbms = refs[1:1 + n_idx]
        out_hbms = refs[1 + n_idx:1 + 2 * n_idx]
        idx_v, rows_v, sem = refs[1 + 2 * n_idx:]
        c = lax.axis_index("c")
        s = lax.axis_index("s")
        base = (s * _NC + c) * epw
        for j in range(n_idx):
            pltpu.sync_copy(idx_hbms[j].at[pl.ds(base, epw)], idx_v)
            pltpu.async_copy(table_hbm.at[idx_v], rows_v, sem).wait()
            pltpu.sync_copy(rows_v, out_hbms[j].at[pl.ds(base, epw)])

    return list(k(table, *idx_list))


def _sc_scatter_add(msg, col, zeros_init):
    e_total = msg.shape[0]
    n_rows, width = zeros_init.shape
    epw = e_total // _NW
    rows_per_tile = n_rows // _NS
    mesh = plsc.VectorSubcoreMesh(core_axis_name="c", subcore_axis_name="s")

    @functools.partial(
        pl.kernel,
        out_type=jax.ShapeDtypeStruct((_NC * n_rows, width), jnp.float32),
        mesh=mesh,
        scratch_types=[
            pltpu.VMEM((epw,), jnp.int32),
            pltpu.VMEM((epw, width), jnp.float32),
            pltpu.VMEM_SHARED((n_rows, width), jnp.float32),
            pltpu.SemaphoreType.DMA,
        ],
        compiler_params=pltpu.CompilerParams(use_tc_tiling_on_sc=False),
    )
    def k(msg_hbm, col_hbm, zero_hbm, out_hbm, idx_v, msg_v, shared, sem):
        c = lax.axis_index("c")
        s = lax.axis_index("s")
        base = (c * _NS + s) * epw

        @pl.when(s == 0)
        def _():
            pltpu.sync_copy(zero_hbm, shared)

        plsc.subcore_barrier()
        pltpu.sync_copy(col_hbm.at[pl.ds(base, epw)], idx_v)
        pltpu.sync_copy(msg_hbm.at[pl.ds(base, epw)], msg_v)
        pltpu.sync_copy(msg_v, shared.at[idx_v], add=True)
        plsc.subcore_barrier()
        pltpu.sync_copy(
            shared.at[pl.ds(s * rows_per_tile, rows_per_tile)],
            out_hbm.at[pl.ds(c * n_rows + s * rows_per_tile, rows_per_tile)],
        )

    return k(msg, col, zeros_init)


def _tc_h0(x_p, w1big, b1rep):
    gn = x_p.shape[0]

    def body(x_ref, w_ref, b_ref, o_ref):
        o_ref[...] = jax.nn.silu(
            jnp.dot(x_ref[...], w_ref[...], preferred_element_type=jnp.float32)
            + b_ref[...])

    return pl.pallas_call(
        body,
        out_shape=jax.ShapeDtypeStruct((gn, 128), jnp.float32),
    )(x_p, w1big, b1rep)


def _tc_z(pr_p, pc_p, ea32, sg, c_ea, c_dist, a1rep):
    g_total = pr_p.shape[0]
    blk = 2000

    def body(pr_ref, pc_ref, ea_ref, sg_ref, cea_ref, cd_ref, a1b_ref, o_ref):
        d = pr_ref[...] - pc_ref[...]
        t8 = jnp.dot(d * d, sg_ref[...], preferred_element_type=jnp.float32)
        dist8 = jnp.sqrt(t8 + 1e-12)
        pre = (jnp.dot(ea_ref[...], cea_ref[...],
                       preferred_element_type=jnp.float32)
               + jnp.dot(dist8, cd_ref[...],
                         preferred_element_type=jnp.float32)
               + a1b_ref[...])
        o_ref[...] = jax.nn.silu(pre)

    return pl.pallas_call(
        body,
        grid=(g_total // blk,),
        in_specs=[
            pl.BlockSpec((blk, 128), lambda i: (i, 0)),
            pl.BlockSpec((blk, 128), lambda i: (i, 0)),
            pl.BlockSpec((blk, 32), lambda i: (i, 0)),
            pl.BlockSpec((128, 8), lambda i: (0, 0)),
            pl.BlockSpec((32, 256), lambda i: (0, 0)),
            pl.BlockSpec((8, 256), lambda i: (0, 0)),
            pl.BlockSpec((1, 256), lambda i: (0, 0)),
        ],
        out_specs=pl.BlockSpec((blk, 256), lambda i: (i, 0)),
        out_shape=jax.ShapeDtypeStruct((g_total, 256), jnp.float32),
    )(pr_p, pc_p, ea32, sg, c_ea, c_dist, a1rep)


def _tc_msg(z_p, u_p, a2bd4, a2rep, tbd, sbd):
    g_total = u_p.shape[0]
    blk = 400

    def body(z_ref, u_ref, a2_ref, a2b_ref, t_ref, s_ref, o_ref):
        zv = z_ref[...]
        wep_a = jnp.dot(zv[:, :128], a2_ref[...],
                        preferred_element_type=jnp.float32)
        wep_b = jnp.dot(zv[:, 128:], a2_ref[...],
                        preferred_element_type=jnp.float32)
        wep = jnp.concatenate([wep_a, wep_b], axis=1) + a2b_ref[...]
        uv = u_ref[...]
        uexp = jnp.concatenate(
            [jnp.tile(uv[:, 16 * j:16 * (j + 1)], (1, 16)) for j in range(8)],
            axis=1)
        o_ref[...] = jnp.dot(wep * uexp, s_ref[...],
                             preferred_element_type=jnp.float32)

    return pl.pallas_call(
        body,
        grid=(g_total // blk,),
        in_specs=[
            pl.BlockSpec((blk, 256), lambda i: (i, 0)),
            pl.BlockSpec((blk, 128), lambda i: (i, 0)),
            pl.BlockSpec((128, 1024), lambda i: (0, 0)),
            pl.BlockSpec((1, 2048), lambda i: (0, 0)),
            pl.BlockSpec((128, 2048), lambda i: (0, 0)),
            pl.BlockSpec((2048, 128), lambda i: (0, 0)),
        ],
        out_specs=pl.BlockSpec((blk, 128), lambda i: (i, 0)),
        out_shape=jax.ShapeDtypeStruct((g_total, 128), jnp.float32),
    )(z_p, u_p, a2bd4, a2rep, tbd, sbd)


def _tc_gru(h_p, parts_p, wr_bd, br_rep, wih_big, whh_big, bih_big, bhh_big):
    gn = h_p.shape[0]

    def body(h_ref, p_ref, wr_ref, br_ref, wih_ref, whh_ref, bih_ref,
             bhh_ref, o_ref):
        hv = h_ref[...]
        aggr = p_ref[:gn, :] + p_ref[gn:, :]
        hc = jax.nn.silu(
            jnp.dot(hv, wr_ref[...], preferred_element_type=jnp.float32)
            + br_ref[...] + aggr)
        gi = jnp.dot(hc, wih_ref[...], preferred_element_type=jnp.float32) + bih_ref[...]
        gh = jnp.dot(hv, whh_ref[...], preferred_element_type=jnp.float32) + bhh_ref[...]
        r = jax.nn.sigmoid(gi[:, 0:128] + gh[:, 0:128])
        z = jax.nn.sigmoid(gi[:, 128:256] + gh[:, 128:256])
        nn = jnp.tanh(gi[:, 256:384] + r * gh[:, 256:384])
        o_ref[...] = (1.0 - z) * nn + z * hv

    return pl.pallas_call(
        body,
        out_shape=jax.ShapeDtypeStruct((gn, 128), jnp.float32),
    )(h_p, parts_p, wr_bd, br_rep, wih_big, whh_big, bih_big, bhh_big)


def _tc_set2set(h_state, batch_col, wi_t, wh_t, bi_row, bh_row,
                o1m, o1b, o2m, o2b, m_steps, n_graphs):
    n, h = h_state.shape

    def body(h_ref, b_ref, wi_ref, wh_ref, bi_ref, bh_ref,
             o1_ref, o1b_ref, o2_ref, o2b_ref, out_ref):
        hv = h_ref[...]
        seg = b_ref[...]
        gid = lax.broadcasted_iota(jnp.int32, (n, n_graphs), 1)
        mask = seg == gid
        onehot = mask.astype(jnp.float32)

        q_star = jnp.zeros((n_graphs, 2 * h), jnp.float32)
        lh = jnp.zeros((n_graphs, h), jnp.float32)
        lc = jnp.zeros((n_graphs, h), jnp.float32)
        for _ in range(m_steps):
            g = (jnp.dot(q_star, wi_ref[...], preferred_element_type=jnp.float32)
                 + bi_ref[...]
                 + jnp.dot(lh, wh_ref[...], preferred_element_type=jnp.float32)
                 + bh_ref[...])
            i_ = jax.nn.sigmoid(g[:, 0:h])
            f_ = jax.nn.sigmoid(g[:, h:2 * h])
            c_ = jnp.tanh(g[:, 2 * h:3 * h])
            o_ = jax.nn.sigmoid(g[:, 3 * h:4 * h])
            lc = f_ * lc + i_ * c_
            lh = o_ * jnp.tanh(lc)
            q = lh
            qpn = jnp.dot(onehot, q, preferred_element_type=jnp.float32)
            e = jnp.sum(hv * qpn, axis=1, keepdims=True)
            me = jnp.where(mask, e, -3e38)
            emax = jnp.max(me, axis=0, keepdims=True)
            emax = jnp.where(emax > -1e38, emax, 0.0)
            eg = jnp.sum(onehot * emax, axis=1, keepdims=True)
            ee = jnp.exp(e - eg)
            den = jnp.sum(onehot * ee, axis=0, keepdims=True)
            dg = jnp.sum(onehot * den, axis=1, keepdims=True)
            a = ee / (dg + 1e-16)
            wgt = onehot * a
            rvec = lax.dot_general(wgt, hv, (((0,), (0,)), ((), ())),
                                   preferred_element_type=jnp.float32)
            q_star = jnp.concatenate([q, rvec], axis=1)

        s = jax.nn.silu(
            jnp.dot(q_star, o1_ref[...], preferred_element_type=jnp.float32)
            + o1b_ref[...])
        out_ref[...] = (jnp.dot(s, o2_ref[...], preferred_element_type=jnp.float32)
                        + o2b_ref[...])

    return pl.pallas_call(
        body,
        out_shape=jax.ShapeDtypeStruct((n_graphs, 1), jnp.float32),
    )(h_state, batch_col, wi_t, wh_t, bi_row, bh_row, o1m, o1b, o2m, o2b)


def kernel(x, edge_index, edge_attr, pos, batch, W1, b1, A1, a1, A2, a2,
           Wr, br, Wih, Whh, bih, bhh, Wi, Wh, bi, bh, O1, o1, O2, o2):
    n, d_in = x.shape
    e_total = edge_index.shape[1]
    h = W1.shape[1]
    m_steps = 3
    num_layers = 4
    b_graphs = 16
    ge = e_total // 8
    gn = n // 8

    row = edge_index[0]
    col = edge_index[1]

    eye8 = np.eye(8, dtype=np.float32)
    eye16 = np.eye(16, dtype=np.float32)
    sg = jnp.asarray(np.kron(eye8, np.ones((16, 1), np.float32)))
    c_ea = jnp.kron(eye8, A1[:4])
    c_dist = jnp.kron(eye8, A1[4:5])
    a1rep = jnp.tile(a1.reshape(1, 2 * h), (1, 8))
    a2perm = A2.reshape(2 * h, h, h).transpose(0, 2, 1).reshape(2 * h, h * h)
    a2bd4 = jnp.kron(jnp.eye(4, dtype=jnp.float32), a2perm)
    a2rep = jnp.tile(a2.reshape(h, h).T.reshape(1, h * h), (1, 8))
    t16 = np.tile(eye16, (1, 16))
    s16 = np.repeat(eye16, 16, axis=0)
    tbd = jnp.asarray(np.kron(eye8, t16))
    sbd = jnp.asarray(np.kron(eye8, s16))
    w1big = jnp.kron(jnp.eye(8, dtype=jnp.float32), W1)
    b1rep = jnp.tile(b1.reshape(1, h), (1, 8))
    wr_bd = jnp.kron(jnp.eye(8, dtype=jnp.float32), Wr)
    br_rep = jnp.tile(br.reshape(1, h), (1, 8))
    wih_t = Wih.T
    whh_t = Whh.T
    wih_big = jnp.concatenate(
        [jnp.kron(jnp.eye(8, dtype=jnp.float32),
                  wih_t[:, 16 * g:16 * (g + 1)]) for g in range(3)], axis=1)
    whh_big = jnp.concatenate(
        [jnp.kron(jnp.eye(8, dtype=jnp.float32),
                  whh_t[:, 16 * g:16 * (g + 1)]) for g in range(3)], axis=1)
    bih_big = jnp.concatenate(
        [jnp.tile(bih[16 * g:16 * (g + 1)].reshape(1, 16), (1, 8))
         for g in range(3)], axis=1)
    bhh_big = jnp.concatenate(
        [jnp.tile(bhh[16 * g:16 * (g + 1)].reshape(1, 16), (1, 8))
         for g in range(3)], axis=1)

    pos16 = jnp.zeros((n, 16), jnp.float32).at[:, :3].set(pos)
    ea32 = edge_attr.reshape(ge, 32)
    zeros_init = jnp.zeros((n, h), jnp.float32)
    x_p = x.reshape(gn, 8 * d_in)

    pr, pc = _sc_gather(pos16, [row, col])
    z_p = _tc_z(pr.reshape(ge, 128), pc.reshape(ge, 128), ea32,
                sg, c_ea, c_dist, a1rep)
    h_p = _tc_h0(x_p, w1big, b1rep)

    for _ in range(num_layers):
        (u,) = _sc_gather(h_p.reshape(n, h), [row])
        msg_p = _tc_msg(z_p, u.reshape(ge, 128), a2bd4, a2rep, tbd, sbd)
        parts = _sc_scatter_add(msg_p.reshape(e_total, h), col, zeros_init)
        h_p = _tc_gru(h_p, parts.reshape(2 * gn, 128), wr_bd, br_rep,
                      wih_big, whh_big, bih_big, bhh_big)

    out = _tc_set2set(h_p.reshape(n, h), batch.reshape(n, 1), Wi.T, Wh.T,
                      bi.reshape(1, 4 * h), bh.reshape(1, 4 * h),
                      O1, o1.reshape(1, h), O2, o2.reshape(1, 1),
                      m_steps, b_graphs)
    return jnp.squeeze(out)
